# Initial kernel scaffold; baseline (speedup 1.0000x reference)
import jax
import jax.numpy as jnp
from jax.experimental import pallas as pl


def kernel(logits, targets, index, delta_smooth):
    raise NotImplementedError("write your pallas kernel here")


# speedup vs baseline: 1.5745x; 1.5745x over previous
import functools

import jax
import jax.numpy as jnp
from jax import lax
from jax.experimental import pallas as pl
from jax.experimental.pallas import tpu as pltpu
from jax.experimental.pallas import tpu_sc as plsc

_SC_NUM_CORES = 2
_SC_NUM_SUBCORES = 16


def _sc_gather(table, idx):
    B = idx.shape[0]
    nw = _SC_NUM_CORES * _SC_NUM_SUBCORES
    b_per_w = B // nw
    mesh = plsc.VectorSubcoreMesh(core_axis_name="c", subcore_axis_name="s")

    @functools.partial(
        pl.kernel,
        mesh=mesh,
        out_type=jax.ShapeDtypeStruct((B,), jnp.float32),
        scratch_types=[
            pltpu.VMEM((b_per_w,), jnp.int32),
            pltpu.VMEM((b_per_w,), jnp.float32),
            pltpu.SemaphoreType.DMA,
        ],
    )
    def gather_kernel(table_hbm, idx_hbm, out_hbm, idx_v, vals_v, sem):
        wid = lax.axis_index("s") * _SC_NUM_CORES + lax.axis_index("c")
        base = wid * b_per_w
        pltpu.sync_copy(idx_hbm.at[pl.ds(base, b_per_w)], idx_v)
        pltpu.async_copy(table_hbm.at[idx_v], vals_v, sem).wait()
        pltpu.sync_copy(vals_v, out_hbm.at[pl.ds(base, b_per_w)])

    return gather_kernel(table, idx)


def _tc_loss(logits, targets2d, ds2d):
    B, C1 = logits.shape
    BR = 512
    inv_b = 1.0 / B

    def body(x_ref, t_ref, d_ref, out_ref):
        i = pl.program_id(0)
        x = x_ref[...]
        m = jnp.max(x, axis=1, keepdims=True)
        s = jnp.sum(jnp.exp(x - m), axis=1, keepdims=True)
        lse = jnp.log(s) + m
        cols = lax.broadcasted_iota(jnp.int32, (BR, C1), 1)
        lt = jnp.sum(jnp.where(cols == t_ref[...], x, 0.0), axis=1,
                     keepdims=True)
        lc = x[:, C1 - 1:C1]
        d = d_ref[...]
        per_row = lse - (1.0 - d) * lt - d * lc

        @pl.when(i == 0)
        def _():
            out_ref[0, 0] = 0.0

        out_ref[0, 0] += jnp.sum(per_row) * inv_b

    return pl.pallas_call(
        body,
        grid=(B // BR,),
        in_specs=[
            pl.BlockSpec((BR, C1), lambda i: (i, 0)),
            pl.BlockSpec((BR, 1), lambda i: (i, 0)),
            pl.BlockSpec((BR, 1), lambda i: (i, 0)),
        ],
        out_specs=pl.BlockSpec(memory_space=pltpu.SMEM),
        out_shape=jax.ShapeDtypeStruct((1, 1), jnp.float32),
    )(logits, targets2d, ds2d)


def kernel(logits, targets, index, delta_smooth):
    B = logits.shape[0]
    ds = _sc_gather(delta_smooth, index.astype(jnp.int32))
    out = _tc_loss(logits, targets.astype(jnp.int32).reshape(B, 1),
                   ds.reshape(B, 1))
    return out[0, 0]


# speedup vs baseline: 2.9526x; 1.8752x over previous
import functools

import jax
import jax.numpy as jnp
from jax import lax
from jax.experimental import pallas as pl
from jax.experimental.pallas import tpu as pltpu
from jax.experimental.pallas import tpu_sc as plsc

_SC_NUM_CORES = 2
_SC_NUM_SUBCORES = 16


def _sc_gather(table, idx):
    B = idx.shape[0]
    nw = _SC_NUM_CORES * _SC_NUM_SUBCORES
    b_per_w = B // nw
    mesh = plsc.VectorSubcoreMesh(core_axis_name="c", subcore_axis_name="s")

    @functools.partial(
        pl.kernel,
        mesh=mesh,
        out_type=jax.ShapeDtypeStruct((B,), jnp.float32),
        scratch_types=[
            pltpu.VMEM((b_per_w,), jnp.int32),
            pltpu.VMEM((b_per_w,), jnp.float32),
            pltpu.SemaphoreType.DMA,
        ],
    )
    def gather_kernel(table_hbm, idx_hbm, out_hbm, idx_v, vals_v, sem):
        wid = lax.axis_index("s") * _SC_NUM_CORES + lax.axis_index("c")
        base = wid * b_per_w
        pltpu.sync_copy(idx_hbm.at[pl.ds(base, b_per_w)], idx_v)
        pltpu.async_copy(table_hbm.at[idx_v], vals_v, sem).wait()
        pltpu.sync_copy(vals_v, out_hbm.at[pl.ds(base, b_per_w)])

    return gather_kernel(table, idx)


def _tc_loss(logits_t, targets3d, ds3d):
    C1, B = logits_t.shape
    BC = 512
    nblk = B // BC
    inv_b = 1.0 / B

    def body(x_ref, t_ref, d_ref, out_ref):
        i = pl.program_id(0)
        x = x_ref[...]
        m = jnp.max(x, axis=0, keepdims=True)
        s = jnp.sum(jnp.exp(x - m), axis=0, keepdims=True)
        lse = jnp.log(s) + m
        rows = lax.broadcasted_iota(jnp.int32, (C1, BC), 0)
        t = t_ref[0]
        lt = jnp.sum(jnp.where(rows == t, x, 0.0), axis=0, keepdims=True)
        lc = x[C1 - 1:C1, :]
        d = d_ref[0]
        per_col = lse - (1.0 - d) * lt - d * lc

        @pl.when(i == 0)
        def _():
            out_ref[0, 0] = 0.0

        out_ref[0, 0] += jnp.sum(per_col) * inv_b

    return pl.pallas_call(
        body,
        grid=(nblk,),
        in_specs=[
            pl.BlockSpec((C1, BC), lambda i: (0, i)),
            pl.BlockSpec((1, 1, BC), lambda i: (i, 0, 0)),
            pl.BlockSpec((1, 1, BC), lambda i: (i, 0, 0)),
        ],
        out_specs=pl.BlockSpec(memory_space=pltpu.SMEM),
        out_shape=jax.ShapeDtypeStruct((1, 1), jnp.float32),
    )(logits_t, targets3d, ds3d)


def kernel(logits, targets, index, delta_smooth):
    B, _ = logits.shape
    BC = 512
    nblk = B // BC
    hbm = pltpu.MemorySpace.HBM
    ds = _sc_gather(delta_smooth, index.astype(jnp.int32))
    logits_t = pltpu.with_memory_space_constraint(logits.T, hbm)
    t3 = pltpu.with_memory_space_constraint(
        targets.astype(jnp.int32).reshape(nblk, 1, BC), hbm)
    d3 = pltpu.with_memory_space_constraint(ds.reshape(nblk, 1, BC), hbm)
    out = _tc_loss(logits_t, t3, d3)
    return out[0, 0]


# speedup vs baseline: 3.4559x; 1.1704x over previous
import functools

import jax
import jax.numpy as jnp
from jax import lax
from jax.experimental import pallas as pl
from jax.experimental.pallas import tpu as pltpu
from jax.experimental.pallas import tpu_sc as plsc

_SC_NUM_CORES = 2
_SC_NUM_SUBCORES = 16


def _sc_gather(table, idx):
    B = idx.shape[0]
    nw = _SC_NUM_CORES * _SC_NUM_SUBCORES
    b_per_w = B // nw
    mesh = plsc.VectorSubcoreMesh(core_axis_name="c", subcore_axis_name="s")

    @functools.partial(
        pl.kernel,
        mesh=mesh,
        out_type=jax.ShapeDtypeStruct((B,), jnp.float32),
        scratch_types=[
            pltpu.VMEM((b_per_w,), jnp.int32),
            pltpu.VMEM((b_per_w,), jnp.float32),
            pltpu.SemaphoreType.DMA,
        ],
    )
    def gather_kernel(table_hbm, idx_hbm, out_hbm, idx_v, vals_v, sem):
        wid = lax.axis_index("s") * _SC_NUM_CORES + lax.axis_index("c")
        base = wid * b_per_w
        pltpu.sync_copy(idx_hbm.at[pl.ds(base, b_per_w)], idx_v)
        pltpu.async_copy(table_hbm.at[idx_v], vals_v, sem).wait()
        pltpu.sync_copy(vals_v, out_hbm.at[pl.ds(base, b_per_w)])

    return gather_kernel(table, idx)


def _tc_main(logits_t, targets3d, bc):
    C1, B = logits_t.shape
    nblk = B // bc

    def body(x_ref, t_ref, u_ref, v_ref):
        x = x_ref[...]
        m = jnp.max(x, axis=0, keepdims=True)
        s = jnp.sum(jnp.exp(x - m), axis=0, keepdims=True)
        lse = jnp.log(s) + m
        rows = lax.broadcasted_iota(jnp.int32, (C1, bc), 0)
        lt = jnp.sum(jnp.where(rows == t_ref[0], x, 0.0), axis=0,
                     keepdims=True)
        lc = x[C1 - 1:C1, :]
        u_ref[0] = lse - lt
        v_ref[0] = lt - lc

    return pl.pallas_call(
        body,
        grid=(nblk,),
        in_specs=[
            pl.BlockSpec((C1, bc), lambda i: (0, i)),
            pl.BlockSpec((1, 1, bc), lambda i: (i, 0, 0)),
        ],
        out_specs=[
            pl.BlockSpec((1, 1, bc), lambda i: (i, 0, 0)),
            pl.BlockSpec((1, 1, bc), lambda i: (i, 0, 0)),
        ],
        out_shape=[
            jax.ShapeDtypeStruct((nblk, 1, bc), jnp.float32),
            jax.ShapeDtypeStruct((nblk, 1, bc), jnp.float32),
        ],
    )(logits_t, targets3d)


def _tc_combine(u3, v3, ds3, inv_b):
    def body(u_ref, v_ref, d_ref, out_ref):
        out_ref[0, 0] = jnp.sum(
            u_ref[...] + d_ref[...] * v_ref[...]) * inv_b

    return pl.pallas_call(
        body,
        out_specs=pl.BlockSpec(memory_space=pltpu.SMEM),
        out_shape=jax.ShapeDtypeStruct((1, 1), jnp.float32),
    )(u3, v3, ds3)


def kernel(logits, targets, index, delta_smooth):
    B, _ = logits.shape
    bc = 1024
    nblk = B // bc
    hbm = pltpu.MemorySpace.HBM
    ds = _sc_gather(delta_smooth, index.astype(jnp.int32))
    logits_t = pltpu.with_memory_space_constraint(logits.T, hbm)
    t3 = pltpu.with_memory_space_constraint(
        targets.astype(jnp.int32).reshape(nblk, 1, bc), hbm)
    u3, v3 = _tc_main(logits_t, t3, bc)
    out = _tc_combine(u3, v3, ds.reshape(nblk, 1, bc), 1.0 / B)
    return out[0, 0]
